# baseline (speedup 1.0000x reference)
import functools

import jax
import jax.numpy as jnp
from jax import lax
from jax.experimental import pallas as pl
from jax.experimental.pallas import tpu as pltpu
from jax.experimental.pallas import tpu_sc as plsc

B, L, WIN, H = 64, 4096, 5, 64
NUM_ST, NUM_IN = 16, 2
SIGMA = 0.5
NTAB = NUM_ST * NUM_IN
D = 2 * B
NW = 32
ROWS_PER_W = L // NW
CH = 512


@functools.lru_cache(maxsize=None)
def _make_sc_gather():
    mesh = plsc.VectorSubcoreMesh(core_axis_name="c", subcore_axis_name="s")

    @functools.partial(
        pl.kernel,
        out_type=jax.ShapeDtypeStruct((L, D), jnp.float32),
        mesh=mesh,
        scratch_types=[
            pltpu.VMEM((ROWS_PER_W,), jnp.int32),
            pltpu.VMEM((ROWS_PER_W, D), jnp.float32),
            pltpu.SemaphoreType.DMA,
        ],
    )
    def sc_gather(table_hbm, idx_hbm, out_hbm, idx_v, rows_v, sem):
        wid = lax.axis_index("s") * 2 + lax.axis_index("c")
        base = wid * ROWS_PER_W
        pltpu.sync_copy(idx_hbm.at[pl.ds(base, ROWS_PER_W)], idx_v)
        pltpu.async_copy(table_hbm.at[idx_v], rows_v, sem).wait()
        pltpu.sync_copy(rows_v, out_hbm.at[pl.ds(base, ROWS_PER_W)])

    return sc_gather


def _chunk_shifted(x, lo, k):
    if lo - k < 0:
        pad = jnp.zeros((B, k - lo), x.dtype)
        return jnp.concatenate([pad, x[:, : CH - (k - lo)]], axis=1)
    return x[:, lo - k: lo - k + CH]


def _tree_lookup(x, lo, t):
    vals = [t[n] for n in range(NTAB)]
    for k in range(WIN):
        m = _chunk_shifted(x, lo, k) != 0
        vals = [jnp.where(m, vals[2 * j + 1], vals[2 * j])
                for j in range(len(vals) // 2)]
    return vals[0]


def _tc_body(bits, g, ns, n1, n2, pi, w1a, b1a, w2a, b2a, w1b, b1b,
             w2b, b2b, o_sys, o_par1, o_isys, o_par2, o_c1, o_c2,
             pa_ref, pb_ref):
    bits_i = bits[...]
    wb = 2.0 * pi[...] - 1.0

    def table(w1, b1, w2, b2):
        h = jnp.tanh(jnp.dot(wb, w1[...],
                             preferred_element_type=jnp.float32) + b1[...])
        t = jnp.tanh(jnp.dot(h, w2[...],
                             preferred_element_type=jnp.float32) + b2[...])
        return t[:, 0]

    ta = table(w1a, b1a, w2a, b2a)
    tb = table(w1b, b1b, w2b, b2b)

    gv = g[...]
    bpf = jnp.swapaxes(gv[:, :B], 0, 1)
    nsp = jnp.swapaxes(gv[:, B:], 0, 1)

    bp_i = bpf.astype(jnp.int32)
    sum1 = sum2 = sq1 = sq2 = jnp.float32(0.0)
    for c in range(L // CH):
        lo = c * CH
        pa = _tree_lookup(bits_i, lo, ta)
        pb = _tree_lookup(bp_i, lo, tb)
        pa_ref[:, lo:lo + CH] = pa
        pb_ref[:, lo:lo + CH] = pb
        sum1 += jnp.sum(pa)
        sq1 += jnp.sum(pa * pa)
        sum2 += jnp.sum(pb)
        sq2 += jnp.sum(pb * pb)
        xbc = 2.0 * bits_i[:, lo:lo + CH].astype(jnp.float32) - 1.0
        o_sys[:, lo:lo + CH] = xbc + SIGMA * ns[:, lo:lo + CH]
        o_isys[:, lo:lo + CH] = (
            2.0 * bpf[:, lo:lo + CH] - 1.0 + SIGMA * nsp[:, lo:lo + CH])

    inv_n = jnp.float32(1.0 / (B * L))
    m1 = sum1 * inv_n
    m2 = sum2 * inv_n
    is1 = lax.rsqrt(jnp.maximum(sq1 * inv_n - m1 * m1, 1e-30))
    is2 = lax.rsqrt(jnp.maximum(sq2 * inv_n - m2 * m2, 1e-30))

    for c in range(L // CH):
        lo = c * CH
        o_par1[:, lo:lo + CH] = (
            (pa_ref[:, lo:lo + CH] - m1) * is1 + SIGMA * n1[:, lo:lo + CH])
        o_par2[:, lo:lo + CH] = (
            (pb_ref[:, lo:lo + CH] - m2) * is2 + SIGMA * n2[:, lo:lo + CH])

    o_c1[...] = jnp.concatenate(
        [wb[:, WIN - 1:WIN], ((ta - m1) * is1)[:, None]], axis=1)
    o_c2[...] = jnp.concatenate(
        [wb[:, WIN - 1:WIN], ((tb - m2) * is2)[:, None]], axis=1)


def _tc_call(bits, g, ns, n1, n2, pi, *weights):
    return pl.pallas_call(
        _tc_body,
        out_shape=[
            jax.ShapeDtypeStruct((B, L), jnp.float32),
            jax.ShapeDtypeStruct((B, L), jnp.float32),
            jax.ShapeDtypeStruct((B, L), jnp.float32),
            jax.ShapeDtypeStruct((B, L), jnp.float32),
            jax.ShapeDtypeStruct((NTAB, 2), jnp.float32),
            jax.ShapeDtypeStruct((NTAB, 2), jnp.float32),
        ],
        scratch_shapes=[
            pltpu.VMEM((B, L), jnp.float32),
            pltpu.VMEM((B, L), jnp.float32),
        ],
    )(bits, g, ns, n1, n2, pi, *weights)


def kernel(input_stream, permutation, W1a, b1a, W2a, b2a, W1b, b1b, W2b, b2b,
           noise_sys, noise_par1, noise_par2, possible_inputs, next_states,
           prev_states):
    bits = input_stream.astype(jnp.int32)
    ns = noise_sys[:, :, 0]
    packed = jnp.concatenate(
        [input_stream.astype(jnp.float32).T, ns.T], axis=1)
    g = _make_sc_gather()(packed, permutation.astype(jnp.int32))
    o_sys, o_par1, o_isys, o_par2, c1, c2 = _tc_call(
        bits, g, ns, noise_par1[:, :, 0], noise_par2[:, :, 0],
        possible_inputs,
        W1a, b1a.reshape(1, H), W2a, b2a.reshape(1, 1),
        W1b, b1b.reshape(1, H), W2b, b2b.reshape(1, 1))
    expand = lambda x: x[:, :, None]
    return (expand(o_sys), expand(o_par1), expand(o_isys), expand(o_par2),
            c1.reshape(NUM_ST, NUM_IN, 2), c2.reshape(NUM_ST, NUM_IN, 2))
